# baseline (speedup 1.0000x reference)
import functools

import jax
import jax.numpy as jnp
from jax.experimental import pallas as pl

ALPHA = 1.702
LIMIT = 7.0


def _moe_body(hs_ref, ri_ref, rw_ref, wgu_ref, bg_ref, bu_ref,
              wd_ref, bd_ref, out_ref):
    e = pl.program_id(0)
    hs = hs_ref[...]
    ri = ri_ref[...]
    rw = rw_ref[...]

    T, E = rw.shape
    lane = jax.lax.broadcasted_iota(jnp.int32, (T, E), 1)
    wt = jnp.sum(jnp.where(lane == ri, rw, 0.0), axis=1, keepdims=True)
    w_col = jnp.where(ri == e, wt, 0.0)

    ED = bg_ref.shape[-1]
    gu = jnp.dot(hs, wgu_ref[0], preferred_element_type=jnp.float32)
    gate = jnp.minimum(gu[:, :ED] + bg_ref[0], LIMIT)
    up = jnp.clip(gu[:, ED:] + bu_ref[0], -LIMIT, LIMIT)
    glu = gate * jax.nn.sigmoid(gate * ALPHA)
    fused = (up + 1.0) * glu

    contrib = jnp.dot((w_col * fused).astype(jnp.bfloat16), wd_ref[0],
                      preferred_element_type=jnp.float32)
    contrib = contrib + w_col * bd_ref[0]

    @pl.when(e == 0)
    def _():
        out_ref[...] = contrib

    @pl.when(e != 0)
    def _():
        out_ref[...] += contrib


def _moe_call(hs, ri, rw, wgu, bg, bu, wd, bd):
    T, H = hs.shape
    E, _, ED2 = wgu.shape
    ED = ED2 // 2
    grid = (E,)
    return pl.pallas_call(
        _moe_body,
        grid=grid,
        in_specs=[
            pl.BlockSpec((T, H), lambda e: (0, 0)),
            pl.BlockSpec((T, 1), lambda e: (0, 0)),
            pl.BlockSpec((T, E), lambda e: (0, 0)),
            pl.BlockSpec((1, H, ED2), lambda e: (e, 0, 0)),
            pl.BlockSpec((1, 1, ED), lambda e: (e, 0, 0)),
            pl.BlockSpec((1, 1, ED), lambda e: (e, 0, 0)),
            pl.BlockSpec((1, ED, H), lambda e: (e, 0, 0)),
            pl.BlockSpec((1, 1, H), lambda e: (e, 0, 0)),
        ],
        out_specs=pl.BlockSpec((T, H), lambda e: (0, 0)),
        out_shape=jax.ShapeDtypeStruct((T, H), jnp.float32),
    )(hs, ri, rw, wgu, bg, bu, wd, bd)


def kernel(hidden_states, router_indices, routing_weights, gate_up_proj,
           gate_up_proj_bias, down_proj, down_proj_bias):
    B, S, H = hidden_states.shape
    E, _, ED2 = gate_up_proj.shape
    ED = ED2 // 2
    T = B * S
    hs = hidden_states.reshape(T, H).astype(jnp.bfloat16)
    ri = router_indices.reshape(T, 1).astype(jnp.int32)
    rw = routing_weights.reshape(T, E)
    wgu = jnp.concatenate(
        [gate_up_proj[:, :, 0::2], gate_up_proj[:, :, 1::2]], axis=-1
    ).astype(jnp.bfloat16)
    bg = gate_up_proj_bias[:, 0::2].reshape(E, 1, ED)
    bu = gate_up_proj_bias[:, 1::2].reshape(E, 1, ED)
    bd = down_proj_bias.reshape(E, 1, H)
    out = _moe_call(hs, ri, rw, wgu, bg, bu,
                    down_proj.astype(jnp.bfloat16), bd)
    return out.reshape(B, S, H)


# speedup vs baseline: 1.1899x; 1.1899x over previous
import functools

import jax
import jax.numpy as jnp
from jax.experimental import pallas as pl
from jax.experimental.pallas import tpu as pltpu

ALPHA = 1.702
LIMIT = 7.0
TM = 128


def _gmm_body(tid_ref, gid_ref, valid_ref, off_ref,
              hs_ref, w_ref, wgu_ref, bg_ref, bu_ref, wd_ref, bd_ref,
              out_ref):
    i = pl.program_id(0)
    g = gid_ref[i]
    mt = tid_ref[i]

    lo = off_ref[g]
    hi = off_ref[g + 1]
    r0 = mt * TM
    row = jax.lax.broadcasted_iota(jnp.int32, (TM, 1), 0) + r0
    active = (row >= lo) & (row < hi) & (valid_ref[i] > 0)
    w_col = jnp.where(active, w_ref[...], 0.0)

    ED = bg_ref.shape[-1]
    gu = jnp.dot(hs_ref[...], wgu_ref[0], preferred_element_type=jnp.float32)
    gate = jnp.minimum(gu[:, :ED] + bg_ref[0], LIMIT)
    up = jnp.clip(gu[:, ED:] + bu_ref[0], -LIMIT, LIMIT)
    glu = gate * jax.nn.sigmoid(gate * ALPHA)
    fused = (up + 1.0) * glu

    contrib = jnp.dot(w_col * fused, wd_ref[0],
                      preferred_element_type=jnp.float32)
    contrib = contrib + w_col * bd_ref[0]

    prev = tid_ref[jnp.maximum(i - 1, 0)]
    first = (i == 0) | (mt != prev)

    @pl.when(first)
    def _():
        out_ref[...] = contrib

    @pl.when(jnp.logical_not(first))
    def _():
        out_ref[...] += contrib


def _gmm_call(tid, gid, valid, off, hs_s, w_s, wgu, bg, bu, wd, bd):
    T, H = hs_s.shape
    E, _, ED2 = wgu.shape
    ED = ED2 // 2
    W = tid.shape[0]
    grid_spec = pltpu.PrefetchScalarGridSpec(
        num_scalar_prefetch=4,
        grid=(W,),
        in_specs=[
            pl.BlockSpec((TM, H), lambda i, t, g, v, o: (t[i], 0)),
            pl.BlockSpec((TM, 1), lambda i, t, g, v, o: (t[i], 0)),
            pl.BlockSpec((1, H, ED2), lambda i, t, g, v, o: (g[i], 0, 0)),
            pl.BlockSpec((1, 1, ED), lambda i, t, g, v, o: (g[i], 0, 0)),
            pl.BlockSpec((1, 1, ED), lambda i, t, g, v, o: (g[i], 0, 0)),
            pl.BlockSpec((1, ED, H), lambda i, t, g, v, o: (g[i], 0, 0)),
            pl.BlockSpec((1, 1, H), lambda i, t, g, v, o: (g[i], 0, 0)),
        ],
        out_specs=pl.BlockSpec((TM, H), lambda i, t, g, v, o: (t[i], 0)),
    )
    return pl.pallas_call(
        _gmm_body,
        grid_spec=grid_spec,
        out_shape=jax.ShapeDtypeStruct((T, H), jnp.float32),
    )(tid, gid, valid, off, hs_s, w_s, wgu, bg, bu, wd, bd)


def _route_metadata(ri_flat, E, T):
    NT = T // TM
    W = NT + E - 1
    counts = jnp.zeros((E,), jnp.int32).at[ri_flat].add(1)
    off = jnp.concatenate([jnp.zeros((1,), jnp.int32),
                           jnp.cumsum(counts).astype(jnp.int32)])
    perm = jnp.argsort(ri_flat, stable=True).astype(jnp.int32)
    first_tile = off[:-1] // TM
    last_tile = (off[1:] - 1) // TM
    gt = jnp.where(counts > 0, last_tile - first_tile + 1, 0).astype(jnp.int32)
    cum = jnp.cumsum(gt)
    total = cum[-1]
    i = jnp.arange(W, dtype=jnp.int32)
    gid = jnp.searchsorted(cum, i, side='right').astype(jnp.int32)
    valid = (i < total).astype(jnp.int32)
    gid_c = jnp.clip(gid, 0, E - 1)
    start = cum[gid_c] - gt[gid_c]
    tid = first_tile[gid_c] + (i - start)
    gid_f = jnp.where(valid > 0, gid_c, E - 1)
    tid_f = jnp.where(valid > 0, tid, NT - 1).astype(jnp.int32)
    return tid_f, gid_f, valid, off, perm


def kernel(hidden_states, router_indices, routing_weights, gate_up_proj,
           gate_up_proj_bias, down_proj, down_proj_bias):
    B, S, H = hidden_states.shape
    E, _, ED2 = gate_up_proj.shape
    ED = ED2 // 2
    T = B * S
    hs = hidden_states.reshape(T, H)
    ri = router_indices.reshape(T).astype(jnp.int32)
    rw = routing_weights.reshape(T, E)

    tid, gid, valid, off, perm = _route_metadata(ri, E, T)

    hs_s = hs[perm]
    wt = jnp.take_along_axis(rw, ri[:, None], axis=1)[:, 0]
    w_s = wt[perm].reshape(T, 1)

    wgu = jnp.concatenate(
        [gate_up_proj[:, :, 0::2], gate_up_proj[:, :, 1::2]], axis=-1)
    bg = gate_up_proj_bias[:, 0::2].reshape(E, 1, ED)
    bu = gate_up_proj_bias[:, 1::2].reshape(E, 1, ED)
    bd = down_proj_bias.reshape(E, 1, H)

    out_s = _gmm_call(tid, gid, valid, off, hs_s, w_s, wgu, bg, bu,
                      down_proj, bd)
    out = jnp.zeros_like(out_s).at[perm].set(out_s)
    return out.reshape(B, S, H)
